# Initial kernel scaffold; baseline (speedup 1.0000x reference)
import jax
import jax.numpy as jnp
from jax.experimental import pallas as pl


def kernel(x, edge_index, W1, b1, W2, b2):
    raise NotImplementedError("write your pallas kernel here")


# speedup vs baseline: 10.6333x; 10.6333x over previous
import functools

import jax
import jax.numpy as jnp
from jax import lax
from jax.experimental import pallas as pl
from jax.experimental.pallas import tpu as pltpu
from jax.experimental.pallas import tpu_sc as plsc

_NC = 2
_NS = 16
_BD = 80
_BS = 80
_ZR = 16
_W = 80


def _chunk(n):
    return ((n + _NS - 1) // _NS + _W - 1) // _W * _W


_DL = 128


def _make_deg(n, e):
    nw = _NC * _NS
    t = e // nw // _BD
    chunk = _chunk(n)
    ksteps = chunk // _W
    mesh = plsc.VectorSubcoreMesh(core_axis_name="c", subcore_axis_name="s")

    @functools.partial(
        pl.kernel,
        out_type=jax.ShapeDtypeStruct((_NC, n, _DL), jnp.float32),
        mesh=mesh,
        scratch_types=[
            pltpu.VMEM((_BD,), jnp.int32),
            pltpu.VMEM((_BD, _DL), jnp.float32),
            pltpu.VMEM_SHARED((n, _DL), jnp.float32),
        ],
    )
    def deg_kernel(dst3, ones, zcol, out, dst_i, ones_v, acc):
        c = lax.axis_index("c")
        s = lax.axis_index("s")
        wid = c * _NS + s
        pltpu.sync_copy(ones, ones_v)
        for k in range(ksteps):
            off = s * chunk + k * _W

            @pl.when(off + _W <= n)
            def _():
                pltpu.sync_copy(zcol, acc.at[pl.ds(off, _W)])

        plsc.subcore_barrier()

        def body(j, carry):
            pltpu.sync_copy(dst3.at[wid, j], dst_i)
            pltpu.sync_copy(ones_v, acc.at[dst_i], add=True)
            return carry

        lax.fori_loop(0, t, body, 0)
        plsc.subcore_barrier()
        for k in range(ksteps):
            off = s * chunk + k * _W

            @pl.when(off + _W <= n)
            def _():
                pltpu.sync_copy(acc.at[pl.ds(off, _W)],
                                out.at[c, pl.ds(off, _W)])

    return deg_kernel


def _make_scatter(n, h, e):
    nw = _NC * _NS
    t = e // nw // _BS
    chunk = _chunk(n)
    ksteps = chunk // _W
    zsteps = chunk // _ZR
    mesh = plsc.VectorSubcoreMesh(core_axis_name="c", subcore_axis_name="s")

    @functools.partial(
        pl.kernel,
        out_type=jax.ShapeDtypeStruct((_NC, n, h), jnp.float32),
        mesh=mesh,
        scratch_types=[
            pltpu.VMEM((_BS,), jnp.int32),
            pltpu.VMEM((_BS,), jnp.int32),
            pltpu.VMEM((_BS, h), jnp.float32),
            pltpu.VMEM_SHARED((n, h), jnp.float32),
            pltpu.SemaphoreType.DMA,
        ],
    )
    def scat_kernel(hs, src3, dst3, zrow, out,
                    src_i, dst_i, rows_v, acc, gsem):
        c = lax.axis_index("c")
        s = lax.axis_index("s")
        wid = c * _NS + s
        for k in range(zsteps):
            off = s * chunk + k * _ZR

            @pl.when(off + _ZR <= n)
            def _():
                pltpu.sync_copy(zrow, acc.at[pl.ds(off, _ZR)])

        plsc.subcore_barrier()

        def body(j, carry):
            pltpu.sync_copy(src3.at[wid, j], src_i)
            pltpu.sync_copy(dst3.at[wid, j], dst_i)
            pltpu.async_copy(hs.at[src_i], rows_v, gsem).wait()
            pltpu.sync_copy(rows_v, acc.at[dst_i], add=True)
            return carry

        lax.fori_loop(0, t, body, 0)
        plsc.subcore_barrier()
        for k in range(ksteps):
            off = s * chunk + k * _W

            @pl.when(off + _W <= n)
            def _():
                pltpu.sync_copy(acc.at[pl.ds(off, _W)],
                                out.at[c, pl.ds(off, _W)])

    return scat_kernel


_BN = 1000


def _k1_body(x_ref, w_ref, degp_ref, h_ref, hs_ref, dinv_ref):
    deg = degp_ref[0, :, 0:1] + degp_ref[1, :, 0:1] + 1.0
    dinv = lax.rsqrt(deg)
    hm = jnp.dot(x_ref[...], w_ref[...], preferred_element_type=jnp.float32)
    h_ref[...] = hm
    hs_ref[...] = hm * dinv
    dinv_ref[...] = dinv


def _k2_body(p_ref, h_ref, dinv_ref, b_ref, w_ref, h2_ref, hs2_ref):
    dinv = dinv_ref[...]
    agg = p_ref[0] + p_ref[1] + dinv * h_ref[...]
    z = jnp.maximum(dinv * agg + b_ref[...], 0.0)
    h2 = jnp.dot(z, w_ref[...], preferred_element_type=jnp.float32)
    h2_ref[...] = h2
    hs2_ref[...] = h2 * dinv


def _k3_body(p_ref, h_ref, dinv_ref, b_ref, o_ref):
    dinv = dinv_ref[...]
    agg = p_ref[0] + p_ref[1] + dinv * h_ref[...]
    o_ref[...] = jnp.maximum(dinv * agg + b_ref[...], 0.0)


def _row_spec(h):
    return pl.BlockSpec((_BN, h), lambda i: (i, 0))


def _full_spec(a, b):
    return pl.BlockSpec((a, b), lambda i: (0, 0))


def kernel(x, edge_index, W1, b1, W2, b2):
    n, d = x.shape
    h = W1.shape[1]
    e = edge_index.shape[1]

    nw = _NC * _NS
    src3 = edge_index[0].reshape(nw, e // nw // _BS, _BS)
    dst3 = edge_index[1].reshape(nw, e // nw // _BS, _BS)
    dst3d = edge_index[1].reshape(nw, e // nw // _BD, _BD)
    ones1 = jnp.ones((_BD, _DL), jnp.float32)
    zcol = jnp.zeros((_W, _DL), jnp.float32)
    zrow = jnp.zeros((_ZR, h), jnp.float32)
    b1r = b1.reshape(1, h)
    b2r = b2.reshape(1, h)

    deg_fn = _make_deg(n, e)
    scat_fn = _make_scatter(n, h, e)

    degp3 = deg_fn(dst3d, ones1, zcol)

    grid = n // _BN
    p_spec = pl.BlockSpec((_NC, _BN, h), lambda i: (0, i, 0))
    dinv_spec = pl.BlockSpec((_BN, 1), lambda i: (i, 0))
    k1 = pl.pallas_call(
        _k1_body,
        grid=(grid,),
        in_specs=[
            _row_spec(d),
            _full_spec(d, h),
            pl.BlockSpec((_NC, _BN, _DL), lambda i: (0, i, 0)),
        ],
        out_specs=[_row_spec(h), _row_spec(h), dinv_spec],
        out_shape=[
            jax.ShapeDtypeStruct((n, h), jnp.float32),
            jax.ShapeDtypeStruct((n, h), jnp.float32),
            jax.ShapeDtypeStruct((n, 1), jnp.float32),
        ],
    )
    h1, hs1, dinv2 = k1(x, W1, degp3)

    P = scat_fn(hs1, src3, dst3, zrow)

    k2 = pl.pallas_call(
        _k2_body,
        grid=(grid,),
        in_specs=[
            p_spec, _row_spec(h), dinv_spec,
            _full_spec(1, h), _full_spec(h, h),
        ],
        out_specs=[_row_spec(h), _row_spec(h)],
        out_shape=[
            jax.ShapeDtypeStruct((n, h), jnp.float32),
            jax.ShapeDtypeStruct((n, h), jnp.float32),
        ],
    )
    h2, hs2 = k2(P, h1, dinv2, b1r, W2)

    Q = scat_fn(hs2, src3, dst3, zrow)

    k3 = pl.pallas_call(
        _k3_body,
        grid=(grid,),
        in_specs=[p_spec, _row_spec(h), dinv_spec, _full_spec(1, h)],
        out_specs=_row_spec(h),
        out_shape=jax.ShapeDtypeStruct((n, h), jnp.float32),
    )
    return k3(Q, h2, dinv2, b2r)


# speedup vs baseline: 15.1583x; 1.4256x over previous
import functools

import jax
import jax.numpy as jnp
from jax import lax
from jax.experimental import pallas as pl
from jax.experimental.pallas import tpu as pltpu
from jax.experimental.pallas import tpu_sc as plsc

_NC = 2
_NS = 16
_BD = 80
_BS = 80
_ZR = 16
_W = 80


def _chunk(n):
    return ((n + _NS - 1) // _NS + _W - 1) // _W * _W


_DL = 128


def _make_deg(n, e):
    nw = _NC * _NS
    t = e // nw // _BD
    chunk = _chunk(n)
    ksteps = chunk // _W
    mesh = plsc.VectorSubcoreMesh(core_axis_name="c", subcore_axis_name="s")

    @functools.partial(
        pl.kernel,
        out_type=jax.ShapeDtypeStruct((_NC, n, _DL), jnp.float32),
        mesh=mesh,
        scratch_types=[
            pltpu.VMEM((t, _BD), jnp.int32),
            pltpu.VMEM((_BD, _DL), jnp.float32),
            pltpu.VMEM_SHARED((n, _DL), jnp.float32),
        ],
    )
    def deg_kernel(dst3, ones, zcol, out, dst_all, ones_v, acc):
        c = lax.axis_index("c")
        s = lax.axis_index("s")
        wid = c * _NS + s
        pltpu.sync_copy(ones, ones_v)
        pltpu.sync_copy(dst3.at[wid], dst_all)
        for k in range(ksteps):
            off = s * chunk + k * _W

            @pl.when(off + _W <= n)
            def _():
                pltpu.sync_copy(zcol, acc.at[pl.ds(off, _W)])

        plsc.subcore_barrier()

        def body(j, carry):
            pltpu.sync_copy(ones_v, acc.at[dst_all.at[j]], add=True)
            return carry

        lax.fori_loop(0, t, body, 0)
        plsc.subcore_barrier()
        for k in range(ksteps):
            off = s * chunk + k * _W

            @pl.when(off + _W <= n)
            def _():
                pltpu.sync_copy(acc.at[pl.ds(off, _W)],
                                out.at[c, pl.ds(off, _W)])

    return deg_kernel


def _make_scatter(n, h, e):
    nw = _NC * _NS
    t = e // nw // _BS
    chunk = _chunk(n)
    ksteps = chunk // _W
    zsteps = chunk // _ZR
    mesh = plsc.VectorSubcoreMesh(core_axis_name="c", subcore_axis_name="s")

    @functools.partial(
        pl.kernel,
        out_type=jax.ShapeDtypeStruct((_NC, n, h), jnp.float32),
        mesh=mesh,
        scratch_types=[
            pltpu.VMEM((t, _BS), jnp.int32),
            pltpu.VMEM((t, _BS), jnp.int32),
            pltpu.VMEM((_BS, h), jnp.float32),
            pltpu.VMEM_SHARED((n, h), jnp.float32),
            pltpu.SemaphoreType.DMA,
        ],
    )
    def scat_kernel(hs, src3, dst3, zrow, out,
                    src_all, dst_all, rows_v, acc, gsem):
        c = lax.axis_index("c")
        s = lax.axis_index("s")
        wid = c * _NS + s
        pltpu.sync_copy(src3.at[wid], src_all)
        pltpu.sync_copy(dst3.at[wid], dst_all)
        for k in range(zsteps):
            off = s * chunk + k * _ZR

            @pl.when(off + _ZR <= n)
            def _():
                pltpu.sync_copy(zrow, acc.at[pl.ds(off, _ZR)])

        plsc.subcore_barrier()

        def body(j, carry):
            pltpu.async_copy(hs.at[src_all.at[j]], rows_v, gsem).wait()
            pltpu.sync_copy(rows_v, acc.at[dst_all.at[j]], add=True)
            return carry

        lax.fori_loop(0, t, body, 0)
        plsc.subcore_barrier()
        for k in range(ksteps):
            off = s * chunk + k * _W

            @pl.when(off + _W <= n)
            def _():
                pltpu.sync_copy(acc.at[pl.ds(off, _W)],
                                out.at[c, pl.ds(off, _W)])

    return scat_kernel


_BN = 1000


def _k1_body(x_ref, w_ref, degp_ref, h_ref, hs_ref, dinv_ref):
    deg = degp_ref[0, :, 0:1] + degp_ref[1, :, 0:1] + 1.0
    dinv = lax.rsqrt(deg)
    hm = jnp.dot(x_ref[...], w_ref[...], preferred_element_type=jnp.float32)
    h_ref[...] = hm
    hs_ref[...] = hm * dinv
    dinv_ref[...] = dinv


def _k2_body(p_ref, h_ref, dinv_ref, b_ref, w_ref, h2_ref, hs2_ref):
    dinv = dinv_ref[...]
    agg = p_ref[0] + p_ref[1] + dinv * h_ref[...]
    z = jnp.maximum(dinv * agg + b_ref[...], 0.0)
    h2 = jnp.dot(z, w_ref[...], preferred_element_type=jnp.float32)
    h2_ref[...] = h2
    hs2_ref[...] = h2 * dinv


def _k3_body(p_ref, h_ref, dinv_ref, b_ref, o_ref):
    dinv = dinv_ref[...]
    agg = p_ref[0] + p_ref[1] + dinv * h_ref[...]
    o_ref[...] = jnp.maximum(dinv * agg + b_ref[...], 0.0)


def _row_spec(h):
    return pl.BlockSpec((_BN, h), lambda i: (i, 0))


def _full_spec(a, b):
    return pl.BlockSpec((a, b), lambda i: (0, 0))


def kernel(x, edge_index, W1, b1, W2, b2):
    n, d = x.shape
    h = W1.shape[1]
    e = edge_index.shape[1]

    nw = _NC * _NS
    src3 = edge_index[0].reshape(nw, e // nw // _BS, _BS)
    dst3 = edge_index[1].reshape(nw, e // nw // _BS, _BS)
    dst3d = edge_index[1].reshape(nw, e // nw // _BD, _BD)
    ones1 = jnp.ones((_BD, _DL), jnp.float32)
    zcol = jnp.zeros((_W, _DL), jnp.float32)
    zrow = jnp.zeros((_ZR, h), jnp.float32)
    b1r = b1.reshape(1, h)
    b2r = b2.reshape(1, h)

    deg_fn = _make_deg(n, e)
    scat_fn = _make_scatter(n, h, e)

    degp3 = deg_fn(dst3d, ones1, zcol)

    grid = n // _BN
    p_spec = pl.BlockSpec((_NC, _BN, h), lambda i: (0, i, 0))
    dinv_spec = pl.BlockSpec((_BN, 1), lambda i: (i, 0))
    k1 = pl.pallas_call(
        _k1_body,
        grid=(grid,),
        in_specs=[
            _row_spec(d),
            _full_spec(d, h),
            pl.BlockSpec((_NC, _BN, _DL), lambda i: (0, i, 0)),
        ],
        out_specs=[_row_spec(h), _row_spec(h), dinv_spec],
        out_shape=[
            jax.ShapeDtypeStruct((n, h), jnp.float32),
            jax.ShapeDtypeStruct((n, h), jnp.float32),
            jax.ShapeDtypeStruct((n, 1), jnp.float32),
        ],
    )
    h1, hs1, dinv2 = k1(x, W1, degp3)

    P = scat_fn(hs1, src3, dst3, zrow)

    k2 = pl.pallas_call(
        _k2_body,
        grid=(grid,),
        in_specs=[
            p_spec, _row_spec(h), dinv_spec,
            _full_spec(1, h), _full_spec(h, h),
        ],
        out_specs=[_row_spec(h), _row_spec(h)],
        out_shape=[
            jax.ShapeDtypeStruct((n, h), jnp.float32),
            jax.ShapeDtypeStruct((n, h), jnp.float32),
        ],
    )
    h2, hs2 = k2(P, h1, dinv2, b1r, W2)

    Q = scat_fn(hs2, src3, dst3, zrow)

    k3 = pl.pallas_call(
        _k3_body,
        grid=(grid,),
        in_specs=[p_spec, _row_spec(h), dinv_spec, _full_spec(1, h)],
        out_specs=_row_spec(h),
        out_shape=jax.ShapeDtypeStruct((n, h), jnp.float32),
    )
    return k3(Q, h2, dinv2, b2r)


# speedup vs baseline: 19.8883x; 1.3120x over previous
import functools

import jax
import jax.numpy as jnp
from jax import lax
from jax.experimental import pallas as pl
from jax.experimental.pallas import tpu as pltpu
from jax.experimental.pallas import tpu_sc as plsc

_NC = 2
_NS = 16
_BD = 80
_BS = 80
_ZR = 16
_NBUF = 2
_W = 80


def _chunk(n):
    return ((n + _NS - 1) // _NS + _W - 1) // _W * _W


_DL = 128


def _make_deg(n, e):
    nw = _NC * _NS
    t = e // nw // _BD
    chunk = _chunk(n)
    ksteps = chunk // _W
    mesh = plsc.VectorSubcoreMesh(core_axis_name="c", subcore_axis_name="s")

    @functools.partial(
        pl.kernel,
        out_type=jax.ShapeDtypeStruct((_NC, n, _DL), jnp.float32),
        mesh=mesh,
        scratch_types=[
            pltpu.VMEM((t, _BD), jnp.int32),
            pltpu.VMEM((_BD, _DL), jnp.float32),
            pltpu.VMEM_SHARED((n, _DL), jnp.float32),
        ],
    )
    def deg_kernel(dst3, ones, zcol, out, dst_all, ones_v, acc):
        c = lax.axis_index("c")
        s = lax.axis_index("s")
        wid = c * _NS + s
        pltpu.sync_copy(ones, ones_v)
        pltpu.sync_copy(dst3.at[wid], dst_all)
        for k in range(ksteps):
            off = s * chunk + k * _W

            @pl.when(off + _W <= n)
            def _():
                pltpu.sync_copy(zcol, acc.at[pl.ds(off, _W)])

        plsc.subcore_barrier()

        def body(j, carry):
            pltpu.sync_copy(ones_v, acc.at[dst_all.at[j]], add=True)
            return carry

        lax.fori_loop(0, t, body, 0)
        plsc.subcore_barrier()
        for k in range(ksteps):
            off = s * chunk + k * _W

            @pl.when(off + _W <= n)
            def _():
                pltpu.sync_copy(acc.at[pl.ds(off, _W)],
                                out.at[c, pl.ds(off, _W)])

    return deg_kernel


def _make_scatter(n, h, e):
    nw = _NC * _NS
    t = e // nw // _BS
    chunk = _chunk(n)
    ksteps = chunk // _W
    zsteps = chunk // _ZR
    mesh = plsc.VectorSubcoreMesh(core_axis_name="c", subcore_axis_name="s")

    nch = 5 if t % 5 == 0 else 1
    tch = t // nch

    @functools.partial(
        pl.kernel,
        out_type=jax.ShapeDtypeStruct((_NC, n, h), jnp.float32),
        mesh=mesh,
        scratch_types=[
            pltpu.VMEM((tch, _BS), jnp.int32),
            pltpu.VMEM((tch, _BS), jnp.int32),
            pltpu.VMEM((_NBUF, _BS, h), jnp.float32),
            pltpu.VMEM_SHARED((n, h), jnp.float32),
        ] + [pltpu.SemaphoreType.DMA] * _NBUF,
    )
    def scat_kernel(hs, src4, dst4, zrow, out,
                    src_ch, dst_ch, rows_v, acc, *gsems):
        c = lax.axis_index("c")
        s = lax.axis_index("s")
        wid = c * _NS + s
        for k in range(zsteps):
            off = s * chunk + k * _ZR

            @pl.when(off + _ZR <= n)
            def _():
                pltpu.sync_copy(zrow, acc.at[pl.ds(off, _ZR)])

        plsc.subcore_barrier()

        ngroups = tch // _NBUF
        rem = tch - ngroups * _NBUF
        for ch in range(nch):
            pltpu.sync_copy(src4.at[wid, ch], src_ch)
            pltpu.sync_copy(dst4.at[wid, ch], dst_ch)
            for b in range(_NBUF):
                pltpu.async_copy(hs.at[src_ch.at[b]], rows_v.at[b],
                                 gsems[b])

            def body(g, carry):
                for b in range(_NBUF):
                    j = g * _NBUF + b
                    pltpu.make_async_copy(
                        hs.at[src_ch.at[j]], rows_v.at[b],
                        gsems[b]).wait()
                    pltpu.sync_copy(rows_v.at[b], acc.at[dst_ch.at[j]],
                                    add=True)

                    @pl.when(j + _NBUF < tch)
                    def _():
                        pltpu.async_copy(hs.at[src_ch.at[j + _NBUF]],
                                         rows_v.at[b], gsems[b])

                return carry

            lax.fori_loop(0, ngroups, body, 0)
            for r in range(rem):
                j = ngroups * _NBUF + r
                b = j % _NBUF
                pltpu.make_async_copy(
                    hs.at[src_ch.at[j]], rows_v.at[b], gsems[b]).wait()
                pltpu.sync_copy(rows_v.at[b], acc.at[dst_ch.at[j]],
                                add=True)
        plsc.subcore_barrier()
        for k in range(ksteps):
            off = s * chunk + k * _W

            @pl.when(off + _W <= n)
            def _():
                pltpu.sync_copy(acc.at[pl.ds(off, _W)],
                                out.at[c, pl.ds(off, _W)])

    return scat_kernel


_BN = 1000


def _k1_body(x_ref, w_ref, degp_ref, h_ref, hs_ref, dinv_ref):
    deg = degp_ref[0, :, 0:1] + degp_ref[1, :, 0:1] + 1.0
    dinv = lax.rsqrt(deg)
    hm = jnp.dot(x_ref[...], w_ref[...], preferred_element_type=jnp.float32)
    h_ref[...] = hm
    hs_ref[...] = hm * dinv
    dinv_ref[...] = dinv


def _k2_body(p_ref, h_ref, dinv_ref, b_ref, w_ref, h2_ref, hs2_ref):
    dinv = dinv_ref[...]
    agg = p_ref[0] + p_ref[1] + dinv * h_ref[...]
    z = jnp.maximum(dinv * agg + b_ref[...], 0.0)
    h2 = jnp.dot(z, w_ref[...], preferred_element_type=jnp.float32)
    h2_ref[...] = h2
    hs2_ref[...] = h2 * dinv


def _k3_body(p_ref, h_ref, dinv_ref, b_ref, o_ref):
    dinv = dinv_ref[...]
    agg = p_ref[0] + p_ref[1] + dinv * h_ref[...]
    o_ref[...] = jnp.maximum(dinv * agg + b_ref[...], 0.0)


def _row_spec(h):
    return pl.BlockSpec((_BN, h), lambda i: (i, 0))


def _full_spec(a, b):
    return pl.BlockSpec((a, b), lambda i: (0, 0))


def kernel(x, edge_index, W1, b1, W2, b2):
    n, d = x.shape
    h = W1.shape[1]
    e = edge_index.shape[1]

    nw = _NC * _NS
    ts = e // nw // _BS
    nch = 5 if ts % 5 == 0 else 1
    src3 = edge_index[0].reshape(nw, nch, ts // nch, _BS)
    dst3 = edge_index[1].reshape(nw, nch, ts // nch, _BS)
    dst3d = edge_index[1].reshape(nw, e // nw // _BD, _BD)
    ones1 = jnp.ones((_BD, _DL), jnp.float32)
    zcol = jnp.zeros((_W, _DL), jnp.float32)
    zrow = jnp.zeros((_ZR, h), jnp.float32)
    b1r = b1.reshape(1, h)
    b2r = b2.reshape(1, h)

    deg_fn = _make_deg(n, e)
    scat_fn = _make_scatter(n, h, e)

    degp3 = deg_fn(dst3d, ones1, zcol)

    grid = n // _BN
    p_spec = pl.BlockSpec((_NC, _BN, h), lambda i: (0, i, 0))
    dinv_spec = pl.BlockSpec((_BN, 1), lambda i: (i, 0))
    k1 = pl.pallas_call(
        _k1_body,
        grid=(grid,),
        in_specs=[
            _row_spec(d),
            _full_spec(d, h),
            pl.BlockSpec((_NC, _BN, _DL), lambda i: (0, i, 0)),
        ],
        out_specs=[_row_spec(h), _row_spec(h), dinv_spec],
        out_shape=[
            jax.ShapeDtypeStruct((n, h), jnp.float32),
            jax.ShapeDtypeStruct((n, h), jnp.float32),
            jax.ShapeDtypeStruct((n, 1), jnp.float32),
        ],
    )
    h1, hs1, dinv2 = k1(x, W1, degp3)

    P = scat_fn(hs1, src3, dst3, zrow)

    k2 = pl.pallas_call(
        _k2_body,
        grid=(grid,),
        in_specs=[
            p_spec, _row_spec(h), dinv_spec,
            _full_spec(1, h), _full_spec(h, h),
        ],
        out_specs=[_row_spec(h), _row_spec(h)],
        out_shape=[
            jax.ShapeDtypeStruct((n, h), jnp.float32),
            jax.ShapeDtypeStruct((n, h), jnp.float32),
        ],
    )
    h2, hs2 = k2(P, h1, dinv2, b1r, W2)

    Q = scat_fn(hs2, src3, dst3, zrow)

    k3 = pl.pallas_call(
        _k3_body,
        grid=(grid,),
        in_specs=[p_spec, _row_spec(h), dinv_spec, _full_spec(1, h)],
        out_specs=_row_spec(h),
        out_shape=jax.ShapeDtypeStruct((n, h), jnp.float32),
    )
    return k3(Q, h2, dinv2, b2r)
